# baseline (speedup 1.0000x reference)
import functools

import jax
import jax.numpy as jnp
from jax import lax
from jax.experimental import pallas as pl
from jax.experimental.pallas import tpu as pltpu
from jax.experimental.pallas import tpu_sc as plsc

B = 4
NCB = 4
N = 16896
K = 8448
M = N - K
DEG = 3
E = N * DEG
NTOT = NCB * N
KI = K - 24
LTB = NCB * KI
TBS = LTB - 24
NITER = 20
W = B * NCB
HALF = N // 2

NB = 4
NBLK = N // NB

_f32 = jnp.float32

_mesh = plsc.VectorSubcoreMesh(core_axis_name="c", subcore_axis_name="s")
_sc_params = pltpu.CompilerParams(needs_layout_passes=False)


def _phi(v):
    v = jnp.clip(v, 8.5e-8, 16.635532)
    return -jnp.log(jnp.tanh(0.5 * v))


def _d_body(llr_ref, scr_ref, out_ref):
    out_ref[...] = llr_ref[...] * (1.0 - 2.0 * scr_ref[...])


def _descramble(llr, scr):
    return pl.pallas_call(
        _d_body,
        out_shape=jax.ShapeDtypeStruct((B, NTOT), _f32),
    )(llr, scr)


def _f0_body(x_ref, m_ref):
    x = x_ref[...]
    mag = _phi(jnp.abs(x))
    m = jnp.where(x < 0.0, -mag, mag)
    m_ref[...] = jnp.broadcast_to(m[None], (DEG, W, NBLK))


def _first_messages(x):
    return pl.pallas_call(
        _f0_body,
        grid=(NB,),
        in_specs=[pl.BlockSpec((W, NBLK), lambda i: (0, i))],
        out_specs=pl.BlockSpec((DEG, W, NBLK), lambda i: (0, 0, i)),
        out_shape=jax.ShapeDtypeStruct((DEG, W, N), _f32),
    )(x)


def _extrinsic(g_ref, m_ref):
    m = m_ref[...]
    mag = jnp.abs(m)
    ext_mag = _phi(g_ref[0] - mag)
    gn = g_ref[1]
    par = gn - 2.0 * jnp.floor(gn * 0.5)
    sign_tot = 1.0 - 2.0 * par
    ext_sign = jnp.where(m < 0.0, -sign_tot, sign_tot)
    return ext_sign * ext_mag


def _f_body(x_ref, g_ref, m_ref, out_ref):
    c2v = _extrinsic(g_ref, m_ref)
    tot = x_ref[...] + c2v[0] + c2v[1] + c2v[2]
    v2c = tot[None] - c2v
    mag = _phi(jnp.abs(v2c))
    out_ref[...] = jnp.where(v2c < 0.0, -mag, mag)


def _bp_step(x, g, m):
    return pl.pallas_call(
        _f_body,
        grid=(NB,),
        in_specs=[
            pl.BlockSpec((W, NBLK), lambda i: (0, i)),
            pl.BlockSpec((2, DEG, W, NBLK), lambda i: (0, 0, 0, i)),
            pl.BlockSpec((DEG, W, NBLK), lambda i: (0, 0, i)),
        ],
        out_specs=pl.BlockSpec((DEG, W, NBLK), lambda i: (0, 0, i)),
        out_shape=jax.ShapeDtypeStruct((DEG, W, N), _f32),
    )(x, g, m)


def _l_body(x_ref, g_ref, m_ref, a_ref, bits_ref, gm_ref):
    i = pl.program_id(0)
    c2v = _extrinsic(g_ref, m_ref)
    tot = x_ref[...] + c2v[0] + c2v[1] + c2v[2]
    bits = jnp.where(tot < 0.0, 1.0, 0.0)
    bits_ref[...] = bits

    @pl.when(i == 0)
    def _():
        gm_ref[...] = jnp.zeros_like(gm_ref)

    @pl.when(i < K // NBLK)
    def _():
        gm_ref[...] += jnp.dot(bits, a_ref[...],
                               preferred_element_type=jnp.float32)


def _finalize(x, g, m, a):
    return pl.pallas_call(
        _l_body,
        grid=(NB,),
        in_specs=[
            pl.BlockSpec((W, NBLK), lambda i: (0, i)),
            pl.BlockSpec((2, DEG, W, NBLK), lambda i: (0, 0, 0, i)),
            pl.BlockSpec((DEG, W, NBLK), lambda i: (0, 0, i)),
            pl.BlockSpec((NBLK, NCB * 24),
                         lambda i: (jnp.minimum(i, K // NBLK - 1), 0)),
        ],
        out_specs=[
            pl.BlockSpec((W, NBLK), lambda i: (0, i)),
            pl.BlockSpec((W, NCB * 24), lambda i: (0, 0)),
        ],
        out_shape=[
            jax.ShapeDtypeStruct((W, N), _f32),
            jax.ShapeDtypeStruct((W, NCB * 24), _f32),
        ],
    )(x, g, m, a)


def _deinterleave(llr_scr, perm_inv):
    @functools.partial(
        pl.kernel,
        out_type=jax.ShapeDtypeStruct((W * N,), _f32),
        mesh=_mesh,
        compiler_params=_sc_params,
        scratch_types=[
            pltpu.VMEM((NTOT,), _f32),
            pltpu.VMEM((HALF,), jnp.int32),
            pltpu.VMEM((HALF,), _f32),
        ],
    )
    def k(llr_ref, perm_ref, x_ref, tbl_v, idx_v, obuf_v):
        w = lax.axis_index("s")
        h = lax.axis_index("c")
        b = w // NCB
        cb = w % NCB
        pltpu.sync_copy(llr_ref.at[pl.ds(b * NTOT, NTOT)], tbl_v)
        pltpu.sync_copy(perm_ref.at[pl.ds(cb * N + h * HALF, HALF)], idx_v)

        unroll = 16
        def body(j, carry):
            for u in range(unroll):
                o = (j * unroll + u) * 16
                ii = idx_v[pl.ds(o, 16)]
                obuf_v[pl.ds(o, 16)] = plsc.load_gather(tbl_v, [ii])
            return carry

        lax.fori_loop(0, HALF // 16 // unroll, body, 0)
        pltpu.sync_copy(obuf_v, x_ref.at[pl.ds(w * N + h * HALF, HALF)])

    return k(llr_scr.reshape(-1), perm_inv).reshape(W, N)


def _cn_exchange(m, cn3):
    @functools.partial(
        pl.kernel,
        out_type=jax.ShapeDtypeStruct((2 * DEG * W * N,), _f32),
        mesh=_mesh,
        compiler_params=_sc_params,
        scratch_types=[
            pltpu.VMEM((E,), _f32),
            pltpu.VMEM((E,), jnp.int32),
            pltpu.VMEM((M,), _f32),
            pltpu.SemaphoreType.DMA,
        ],
    )
    def k(m_ref, cn_ref, g_ref, dat_v, idx_v, tbl_v, sem):
        q = lax.axis_index("c")
        w = lax.axis_index("s")
        cops = []
        for d in range(DEG):
            off = (d * W + w) * N
            cops.append(pltpu.async_copy(
                m_ref.at[pl.ds(off, N)], dat_v.at[pl.ds(d * N, N)], sem))
        cops.append(pltpu.async_copy(cn_ref, idx_v, sem))

        zunroll = 16
        def zbody(j, carry):
            for u in range(zunroll):
                tbl_v[pl.ds((j * zunroll + u) * 16, 16)] = jnp.zeros((16,), _f32)
            return carry

        lax.fori_loop(0, M // 16 // zunroll, zbody, 0)
        for c in cops:
            c.wait()

        is_mag = q == 0
        one = jnp.ones((16,), _f32)
        zero = jnp.zeros((16,), _f32)

        unroll = 12
        def sbody(j, carry):
            os = [(j * unroll + u) * 16 for u in range(unroll)]
            iis = [idx_v[pl.ds(o, 16)] for o in os]
            mms = [dat_v[pl.ds(o, 16)] for o in os]
            vvs = [jnp.where(is_mag, jnp.abs(mm),
                             jnp.where(mm < 0.0, one, zero)) for mm in mms]
            for ii, vv in zip(iis, vvs):
                plsc.addupdate_scatter(tbl_v, [ii], vv)
            return carry

        lax.fori_loop(0, E // 16 // unroll, sbody, 0)

        ocops = []
        for d in range(DEG):
            def gbody(j, carry, d=d):
                os = [d * N + (j * unroll + u) * 16 for u in range(unroll)]
                iis = [idx_v[pl.ds(o, 16)] for o in os]
                ggs = [plsc.load_gather(tbl_v, [ii]) for ii in iis]
                for o, gg in zip(os, ggs):
                    dat_v[pl.ds(o, 16)] = gg
                return carry

            lax.fori_loop(0, N // 16 // unroll, gbody, 0)
            off = ((q * DEG + d) * W + w) * N
            ocops.append(pltpu.async_copy(
                dat_v.at[pl.ds(d * N, N)], g_ref.at[pl.ds(off, N)], sem))
        for c in ocops:
            c.wait()

    return k(m.reshape(-1), cn3).reshape(2, DEG, W, N)


def kernel(inputs, scr_bits, crc_R, perm_inv, edge_vn, edge_cn):
    del edge_vn
    llr = inputs.astype(_f32).reshape(B, NTOT)
    scr = scr_bits.astype(_f32).reshape(1, NTOT)
    cn3 = edge_cn.reshape(N, DEG).T.reshape(E).astype(jnp.int32)

    llr_scr = _descramble(llr, scr)
    x = _deinterleave(llr_scr, perm_inv.astype(jnp.int32))
    m = _first_messages(x)
    for _ in range(NITER - 1):
        g = _cn_exchange(m, cn3)
        m = _bp_step(x, g, m)
    g = _cn_exchange(m, cn3)

    r4 = crc_R.astype(_f32).reshape(NCB, KI, 24)
    a = jnp.transpose(r4, (1, 0, 2)).reshape(KI, NCB * 24)
    a = jnp.pad(a, ((0, K - KI), (0, 0)))

    bits, gm = _finalize(x, g, m, a)

    gmr = gm.reshape(B, NCB, NCB, 24)
    syn = gmr[:, 0, 0, :] + gmr[:, 1, 1, :] + gmr[:, 2, 2, :] + gmr[:, 3, 3, :]
    syn = syn - 2.0 * jnp.floor(syn * 0.5)
    tb_crc_status = jnp.all(syn < 0.5, axis=-1)
    u_hat = bits[:, :KI].reshape(B, NCB * KI)[:, :TBS]
    return u_hat, tb_crc_status


# speedup vs baseline: 1.0429x; 1.0429x over previous
import functools

import jax
import jax.numpy as jnp
from jax import lax
from jax.experimental import pallas as pl
from jax.experimental.pallas import tpu as pltpu
from jax.experimental.pallas import tpu_sc as plsc

B = 4
NCB = 4
N = 16896
K = 8448
M = N - K
DEG = 3
E = N * DEG
NTOT = NCB * N
KI = K - 24
LTB = NCB * KI
TBS = LTB - 24
NITER = 20
W = B * NCB
HALF = N // 2

NB = 4
NBLK = N // NB

_f32 = jnp.float32

_mesh = plsc.VectorSubcoreMesh(core_axis_name="c", subcore_axis_name="s")
_sc_params = pltpu.CompilerParams(needs_layout_passes=False)


def _phi(v):
    v = jnp.clip(v, 8.5e-8, 16.635532)
    return -jnp.log(jnp.tanh(0.5 * v))


def _d_body(llr_ref, scr_ref, out_ref):
    out_ref[...] = llr_ref[...] * (1.0 - 2.0 * scr_ref[...])


def _descramble(llr, scr):
    return pl.pallas_call(
        _d_body,
        out_shape=jax.ShapeDtypeStruct((B, NTOT), _f32),
    )(llr, scr)


def _f0_body(x_ref, m_ref):
    x = x_ref[...]
    mag = _phi(jnp.abs(x))
    m = jnp.where(x < 0.0, -mag, mag)
    m_ref[...] = jnp.broadcast_to(m[None], (DEG, W, N))


def _first_messages(x):
    return pl.pallas_call(
        _f0_body,
        out_shape=jax.ShapeDtypeStruct((DEG, W, N), _f32),
    )(x)


def _extrinsic(g_ref, m_ref):
    m = m_ref[...]
    mag = jnp.abs(m)
    ext_mag = _phi(g_ref[0] - mag)
    gn = g_ref[1]
    par = gn - 2.0 * jnp.floor(gn * 0.5)
    sign_tot = 1.0 - 2.0 * par
    ext_sign = jnp.where(m < 0.0, -sign_tot, sign_tot)
    return ext_sign * ext_mag


def _f_body(x_ref, g_ref, m_ref, out_ref):
    c2v = _extrinsic(g_ref, m_ref)
    tot = x_ref[...] + c2v[0] + c2v[1] + c2v[2]
    v2c = tot[None] - c2v
    mag = _phi(jnp.abs(v2c))
    out_ref[...] = jnp.where(v2c < 0.0, -mag, mag)


def _bp_step(x, g, m):
    return pl.pallas_call(
        _f_body,
        out_shape=jax.ShapeDtypeStruct((DEG, W, N), _f32),
    )(x, g, m)


def _l_body(x_ref, g_ref, m_ref, a_ref, bits_ref, gm_ref):
    c2v = _extrinsic(g_ref, m_ref)
    tot = x_ref[...] + c2v[0] + c2v[1] + c2v[2]
    bits = jnp.where(tot < 0.0, 1.0, 0.0)
    bits_ref[...] = bits
    gm_ref[...] = jnp.dot(bits[:, :K], a_ref[...],
                          preferred_element_type=jnp.float32)


def _finalize(x, g, m, a):
    return pl.pallas_call(
        _l_body,
        out_shape=[
            jax.ShapeDtypeStruct((W, N), _f32),
            jax.ShapeDtypeStruct((W, NCB * 24), _f32),
        ],
    )(x, g, m, a)


def _deinterleave(llr_scr, perm_inv):
    @functools.partial(
        pl.kernel,
        out_type=jax.ShapeDtypeStruct((W * N,), _f32),
        mesh=_mesh,
        compiler_params=_sc_params,
        scratch_types=[
            pltpu.VMEM((NTOT,), _f32),
            pltpu.VMEM((HALF,), jnp.int32),
            pltpu.VMEM((HALF,), _f32),
        ],
    )
    def k(llr_ref, perm_ref, x_ref, tbl_v, idx_v, obuf_v):
        w = lax.axis_index("s")
        h = lax.axis_index("c")
        b = w // NCB
        cb = w % NCB
        pltpu.sync_copy(llr_ref.at[pl.ds(b * NTOT, NTOT)], tbl_v)
        pltpu.sync_copy(perm_ref.at[pl.ds(cb * N + h * HALF, HALF)], idx_v)

        unroll = 16
        def body(j, carry):
            for u in range(unroll):
                o = (j * unroll + u) * 16
                ii = idx_v[pl.ds(o, 16)]
                obuf_v[pl.ds(o, 16)] = plsc.load_gather(tbl_v, [ii])
            return carry

        lax.fori_loop(0, HALF // 16 // unroll, body, 0)
        pltpu.sync_copy(obuf_v, x_ref.at[pl.ds(w * N + h * HALF, HALF)])

    return k(llr_scr.reshape(-1), perm_inv).reshape(W, N)


def _cn_exchange(m, cn3):
    @functools.partial(
        pl.kernel,
        out_type=jax.ShapeDtypeStruct((2 * DEG * W * N,), _f32),
        mesh=_mesh,
        compiler_params=_sc_params,
        scratch_types=[
            pltpu.VMEM((E,), _f32),
            pltpu.VMEM((E,), jnp.int32),
            pltpu.VMEM((M,), _f32),
            pltpu.SemaphoreType.DMA,
        ],
    )
    def k(m_ref, cn_ref, g_ref, dat_v, idx_v, tbl_v, sem):
        q = lax.axis_index("c")
        w = lax.axis_index("s")
        cops = []
        for d in range(DEG):
            off = (d * W + w) * N
            cops.append(pltpu.async_copy(
                m_ref.at[pl.ds(off, N)], dat_v.at[pl.ds(d * N, N)], sem))
        cops.append(pltpu.async_copy(cn_ref, idx_v, sem))

        zunroll = 16
        def zbody(j, carry):
            for u in range(zunroll):
                tbl_v[pl.ds((j * zunroll + u) * 16, 16)] = jnp.zeros((16,), _f32)
            return carry

        lax.fori_loop(0, M // 16 // zunroll, zbody, 0)
        for c in cops:
            c.wait()

        is_mag = q == 0
        one = jnp.ones((16,), _f32)
        zero = jnp.zeros((16,), _f32)

        unroll = 8
        def sbody(j, carry):
            os = [(j * unroll + u) * 16 for u in range(unroll)]
            iis = [idx_v[pl.ds(o, 16)] for o in os]
            mms = [dat_v[pl.ds(o, 16)] for o in os]
            vvs = [jnp.where(is_mag, jnp.abs(mm),
                             jnp.where(mm < 0.0, one, zero)) for mm in mms]
            for ii, vv in zip(iis, vvs):
                plsc.addupdate_scatter(tbl_v, [ii], vv)
            return carry

        lax.fori_loop(0, E // 16 // unroll, sbody, 0)

        ocops = []
        for d in range(DEG):
            def gbody(j, carry, d=d):
                os = [d * N + (j * unroll + u) * 16 for u in range(unroll)]
                iis = [idx_v[pl.ds(o, 16)] for o in os]
                ggs = [plsc.load_gather(tbl_v, [ii]) for ii in iis]
                for o, gg in zip(os, ggs):
                    dat_v[pl.ds(o, 16)] = gg
                return carry

            lax.fori_loop(0, N // 16 // unroll, gbody, 0)
            off = ((q * DEG + d) * W + w) * N
            ocops.append(pltpu.async_copy(
                dat_v.at[pl.ds(d * N, N)], g_ref.at[pl.ds(off, N)], sem))
        for c in ocops:
            c.wait()

    return k(m.reshape(-1), cn3).reshape(2, DEG, W, N)


def kernel(inputs, scr_bits, crc_R, perm_inv, edge_vn, edge_cn):
    del edge_vn
    llr = inputs.astype(_f32).reshape(B, NTOT)
    scr = scr_bits.astype(_f32).reshape(1, NTOT)
    cn3 = edge_cn.reshape(N, DEG).T.reshape(E).astype(jnp.int32)

    llr_scr = _descramble(llr, scr)
    x = _deinterleave(llr_scr, perm_inv.astype(jnp.int32))
    m = _first_messages(x)
    for _ in range(NITER - 1):
        g = _cn_exchange(m, cn3)
        m = _bp_step(x, g, m)
    g = _cn_exchange(m, cn3)

    r4 = crc_R.astype(_f32).reshape(NCB, KI, 24)
    a = jnp.transpose(r4, (1, 0, 2)).reshape(KI, NCB * 24)
    a = jnp.pad(a, ((0, K - KI), (0, 0)))

    bits, gm = _finalize(x, g, m, a)

    gmr = gm.reshape(B, NCB, NCB, 24)
    syn = gmr[:, 0, 0, :] + gmr[:, 1, 1, :] + gmr[:, 2, 2, :] + gmr[:, 3, 3, :]
    syn = syn - 2.0 * jnp.floor(syn * 0.5)
    tb_crc_status = jnp.all(syn < 0.5, axis=-1)
    u_hat = bits[:, :KI].reshape(B, NCB * KI)[:, :TBS]
    return u_hat, tb_crc_status


# speedup vs baseline: 1.0789x; 1.0345x over previous
import functools

import jax
import jax.numpy as jnp
from jax import lax
from jax.experimental import pallas as pl
from jax.experimental.pallas import tpu as pltpu
from jax.experimental.pallas import tpu_sc as plsc

B = 4
NCB = 4
N = 16896
K = 8448
M = N - K
DEG = 3
E = N * DEG
NTOT = NCB * N
KI = K - 24
LTB = NCB * KI
TBS = LTB - 24
NITER = 20
W = B * NCB
HALF = N // 2

NB = 4
NBLK = N // NB

_f32 = jnp.float32

_mesh = plsc.VectorSubcoreMesh(core_axis_name="c", subcore_axis_name="s")
_sc_params = pltpu.CompilerParams(needs_layout_passes=False)


def _phi(v):
    v = jnp.clip(v, 8.5e-8, 16.635532)
    return -jnp.log(jnp.tanh(0.5 * v))


def _d_body(llr_ref, scr_ref, out_ref):
    out_ref[...] = llr_ref[...] * (1.0 - 2.0 * scr_ref[...])


def _descramble(llr, scr):
    return pl.pallas_call(
        _d_body,
        out_shape=jax.ShapeDtypeStruct((B, NTOT), _f32),
    )(llr, scr)


def _f0_body(x_ref, m_ref):
    x = x_ref[...]
    mag = _phi(jnp.abs(x))
    m = jnp.where(x < 0.0, -mag, mag)
    m_ref[...] = jnp.broadcast_to(m[None], (DEG, W, NBLK))


def _first_messages(x):
    return pl.pallas_call(
        _f0_body,
        grid=(NB,),
        in_specs=[pl.BlockSpec((W, NBLK), lambda i: (0, i))],
        out_specs=pl.BlockSpec((DEG, W, NBLK), lambda i: (0, 0, i)),
        out_shape=jax.ShapeDtypeStruct((DEG, W, N), _f32),
    )(x)


def _extrinsic(g_ref, m_ref):
    m = m_ref[...]
    mag = jnp.abs(m)
    ext_mag = _phi(g_ref[0] - mag)
    gn = g_ref[1]
    par = gn - 2.0 * jnp.floor(gn * 0.5)
    sign_tot = 1.0 - 2.0 * par
    ext_sign = jnp.where(m < 0.0, -sign_tot, sign_tot)
    return ext_sign * ext_mag


def _f_body(x_ref, g_ref, m_ref, out_ref):
    c2v = _extrinsic(g_ref, m_ref)
    tot = x_ref[...] + c2v[0] + c2v[1] + c2v[2]
    v2c = tot[None] - c2v
    mag = _phi(jnp.abs(v2c))
    out_ref[...] = jnp.where(v2c < 0.0, -mag, mag)


def _bp_step(x, g, m):
    return pl.pallas_call(
        _f_body,
        grid=(NB,),
        in_specs=[
            pl.BlockSpec((W, NBLK), lambda i: (0, i)),
            pl.BlockSpec((2, DEG, W, NBLK), lambda i: (0, 0, 0, i)),
            pl.BlockSpec((DEG, W, NBLK), lambda i: (0, 0, i)),
        ],
        out_specs=pl.BlockSpec((DEG, W, NBLK), lambda i: (0, 0, i)),
        out_shape=jax.ShapeDtypeStruct((DEG, W, N), _f32),
    )(x, g, m)


def _l_body(x_ref, g_ref, m_ref, a_ref, bits_ref, gm_ref):
    i = pl.program_id(0)
    c2v = _extrinsic(g_ref, m_ref)
    tot = x_ref[...] + c2v[0] + c2v[1] + c2v[2]
    bits = jnp.where(tot < 0.0, 1.0, 0.0)
    bits_ref[...] = bits

    @pl.when(i == 0)
    def _():
        gm_ref[...] = jnp.zeros_like(gm_ref)

    @pl.when(i < K // NBLK)
    def _():
        gm_ref[...] += jnp.dot(bits, a_ref[...],
                               preferred_element_type=jnp.float32)


def _finalize(x, g, m, a):
    return pl.pallas_call(
        _l_body,
        grid=(NB,),
        in_specs=[
            pl.BlockSpec((W, NBLK), lambda i: (0, i)),
            pl.BlockSpec((2, DEG, W, NBLK), lambda i: (0, 0, 0, i)),
            pl.BlockSpec((DEG, W, NBLK), lambda i: (0, 0, i)),
            pl.BlockSpec((NBLK, NCB * 24),
                         lambda i: (jnp.minimum(i, K // NBLK - 1), 0)),
        ],
        out_specs=[
            pl.BlockSpec((W, NBLK), lambda i: (0, i)),
            pl.BlockSpec((W, NCB * 24), lambda i: (0, 0)),
        ],
        out_shape=[
            jax.ShapeDtypeStruct((W, N), _f32),
            jax.ShapeDtypeStruct((W, NCB * 24), _f32),
        ],
    )(x, g, m, a)


def _deinterleave(llr_scr, perm_inv):
    @functools.partial(
        pl.kernel,
        out_type=jax.ShapeDtypeStruct((W * N,), _f32),
        mesh=_mesh,
        compiler_params=_sc_params,
        scratch_types=[
            pltpu.VMEM((NTOT,), _f32),
            pltpu.VMEM((HALF,), jnp.int32),
            pltpu.VMEM((HALF,), _f32),
        ],
    )
    def k(llr_ref, perm_ref, x_ref, tbl_v, idx_v, obuf_v):
        w = lax.axis_index("s")
        h = lax.axis_index("c")
        b = w // NCB
        cb = w % NCB
        pltpu.sync_copy(llr_ref.at[pl.ds(b * NTOT, NTOT)], tbl_v)
        pltpu.sync_copy(perm_ref.at[pl.ds(cb * N + h * HALF, HALF)], idx_v)

        unroll = 16
        def body(j, carry):
            for u in range(unroll):
                o = (j * unroll + u) * 16
                ii = idx_v[pl.ds(o, 16)]
                obuf_v[pl.ds(o, 16)] = plsc.load_gather(tbl_v, [ii])
            return carry

        lax.fori_loop(0, HALF // 16 // unroll, body, 0)
        pltpu.sync_copy(obuf_v, x_ref.at[pl.ds(w * N + h * HALF, HALF)])

    return k(llr_scr.reshape(-1), perm_inv).reshape(W, N)


def _cn_exchange(m, cn3):
    @functools.partial(
        pl.kernel,
        out_type=jax.ShapeDtypeStruct((2 * DEG * W * N,), _f32),
        mesh=_mesh,
        compiler_params=_sc_params,
        scratch_types=[
            pltpu.VMEM((E,), _f32),
            pltpu.VMEM((E,), jnp.int32),
            pltpu.VMEM((M,), _f32),
            pltpu.SemaphoreType.DMA,
        ],
    )
    def k(m_ref, cn_ref, g_ref, dat_v, idx_v, tbl_v, sem):
        q = lax.axis_index("c")
        w = lax.axis_index("s")
        cops = []
        for d in range(DEG):
            off = (d * W + w) * N
            cops.append(pltpu.async_copy(
                m_ref.at[pl.ds(off, N)], dat_v.at[pl.ds(d * N, N)], sem))
        cops.append(pltpu.async_copy(cn_ref, idx_v, sem))

        zunroll = 16
        def zbody(j, carry):
            for u in range(zunroll):
                tbl_v[pl.ds((j * zunroll + u) * 16, 16)] = jnp.zeros((16,), _f32)
            return carry

        lax.fori_loop(0, M // 16 // zunroll, zbody, 0)
        for c in cops:
            c.wait()

        is_mag = q == 0
        one = jnp.ones((16,), _f32)
        zero = jnp.zeros((16,), _f32)

        unroll = 8
        def sbody(j, carry):
            os = [(j * unroll + u) * 16 for u in range(unroll)]
            iis = [idx_v[pl.ds(o, 16)] for o in os]
            mms = [dat_v[pl.ds(o, 16)] for o in os]
            vvs = [jnp.where(is_mag, jnp.abs(mm),
                             jnp.where(mm < 0.0, one, zero)) for mm in mms]
            for ii, vv in zip(iis, vvs):
                plsc.addupdate_scatter(tbl_v, [ii], vv)
            return carry

        lax.fori_loop(0, E // 16 // unroll, sbody, 0)

        ocops = []
        for d in range(DEG):
            def gbody(j, carry, d=d):
                os = [d * N + (j * unroll + u) * 16 for u in range(unroll)]
                iis = [idx_v[pl.ds(o, 16)] for o in os]
                ggs = [plsc.load_gather(tbl_v, [ii]) for ii in iis]
                for o, gg in zip(os, ggs):
                    dat_v[pl.ds(o, 16)] = gg
                return carry

            lax.fori_loop(0, N // 16 // unroll, gbody, 0)
            off = ((q * DEG + d) * W + w) * N
            ocops.append(pltpu.async_copy(
                dat_v.at[pl.ds(d * N, N)], g_ref.at[pl.ds(off, N)], sem))
        for c in ocops:
            c.wait()

    return k(m.reshape(-1), cn3).reshape(2, DEG, W, N)


def kernel(inputs, scr_bits, crc_R, perm_inv, edge_vn, edge_cn):
    del edge_vn
    llr = inputs.astype(_f32).reshape(B, NTOT)
    scr = scr_bits.astype(_f32).reshape(1, NTOT)
    cn3 = edge_cn.reshape(N, DEG).T.reshape(E).astype(jnp.int32)

    llr_scr = _descramble(llr, scr)
    x = _deinterleave(llr_scr, perm_inv.astype(jnp.int32))
    m = _first_messages(x)
    for _ in range(NITER - 1):
        g = _cn_exchange(m, cn3)
        m = _bp_step(x, g, m)
    g = _cn_exchange(m, cn3)

    r4 = crc_R.astype(_f32).reshape(NCB, KI, 24)
    a = jnp.transpose(r4, (1, 0, 2)).reshape(KI, NCB * 24)
    a = jnp.pad(a, ((0, K - KI), (0, 0)))

    bits, gm = _finalize(x, g, m, a)

    gmr = gm.reshape(B, NCB, NCB, 24)
    syn = gmr[:, 0, 0, :] + gmr[:, 1, 1, :] + gmr[:, 2, 2, :] + gmr[:, 3, 3, :]
    syn = syn - 2.0 * jnp.floor(syn * 0.5)
    tb_crc_status = jnp.all(syn < 0.5, axis=-1)
    u_hat = bits[:, :KI].reshape(B, NCB * KI)[:, :TBS]
    return u_hat, tb_crc_status


# speedup vs baseline: 1.0871x; 1.0076x over previous
import functools

import jax
import jax.numpy as jnp
from jax import lax
from jax.experimental import pallas as pl
from jax.experimental.pallas import tpu as pltpu
from jax.experimental.pallas import tpu_sc as plsc

B = 4
NCB = 4
N = 16896
K = 8448
M = N - K
DEG = 3
E = N * DEG
NTOT = NCB * N
KI = K - 24
LTB = NCB * KI
TBS = LTB - 24
NITER = 20
W = B * NCB
HALF = N // 2

NB = 2
NBLK = N // NB

_f32 = jnp.float32

_mesh = plsc.VectorSubcoreMesh(core_axis_name="c", subcore_axis_name="s")
_sc_params = pltpu.CompilerParams(needs_layout_passes=False)


def _phi(v):
    v = jnp.clip(v, 8.5e-8, 16.635532)
    return -jnp.log(jnp.tanh(0.5 * v))


def _d_body(llr_ref, scr_ref, out_ref):
    out_ref[...] = llr_ref[...] * (1.0 - 2.0 * scr_ref[...])


def _descramble(llr, scr):
    return pl.pallas_call(
        _d_body,
        out_shape=jax.ShapeDtypeStruct((B, NTOT), _f32),
    )(llr, scr)


def _f0_body(x_ref, m_ref):
    x = x_ref[...]
    mag = _phi(jnp.abs(x))
    m = jnp.where(x < 0.0, -mag, mag)
    m_ref[...] = jnp.broadcast_to(m[None], (DEG, W, NBLK))


def _first_messages(x):
    return pl.pallas_call(
        _f0_body,
        grid=(NB,),
        in_specs=[pl.BlockSpec((W, NBLK), lambda i: (0, i))],
        out_specs=pl.BlockSpec((DEG, W, NBLK), lambda i: (0, 0, i)),
        out_shape=jax.ShapeDtypeStruct((DEG, W, N), _f32),
    )(x)


def _extrinsic(g_ref, m_ref):
    m = m_ref[...]
    mag = jnp.abs(m)
    ext_mag = _phi(g_ref[0] - mag)
    gn = g_ref[1]
    par = gn - 2.0 * jnp.floor(gn * 0.5)
    sign_tot = 1.0 - 2.0 * par
    ext_sign = jnp.where(m < 0.0, -sign_tot, sign_tot)
    return ext_sign * ext_mag


def _f_body(x_ref, g_ref, m_ref, out_ref):
    c2v = _extrinsic(g_ref, m_ref)
    tot = x_ref[...] + c2v[0] + c2v[1] + c2v[2]
    v2c = tot[None] - c2v
    mag = _phi(jnp.abs(v2c))
    out_ref[...] = jnp.where(v2c < 0.0, -mag, mag)


def _bp_step(x, g, m):
    return pl.pallas_call(
        _f_body,
        grid=(NB,),
        in_specs=[
            pl.BlockSpec((W, NBLK), lambda i: (0, i)),
            pl.BlockSpec((2, DEG, W, NBLK), lambda i: (0, 0, 0, i)),
            pl.BlockSpec((DEG, W, NBLK), lambda i: (0, 0, i)),
        ],
        out_specs=pl.BlockSpec((DEG, W, NBLK), lambda i: (0, 0, i)),
        out_shape=jax.ShapeDtypeStruct((DEG, W, N), _f32),
    )(x, g, m)


def _l_body(x_ref, g_ref, m_ref, a_ref, bits_ref, gm_ref):
    i = pl.program_id(0)
    c2v = _extrinsic(g_ref, m_ref)
    tot = x_ref[...] + c2v[0] + c2v[1] + c2v[2]
    bits = jnp.where(tot < 0.0, 1.0, 0.0)
    bits_ref[...] = bits

    @pl.when(i == 0)
    def _():
        gm_ref[...] = jnp.zeros_like(gm_ref)

    @pl.when(i < K // NBLK)
    def _():
        gm_ref[...] += jnp.dot(bits, a_ref[...],
                               preferred_element_type=jnp.float32)


def _finalize(x, g, m, a):
    return pl.pallas_call(
        _l_body,
        grid=(NB,),
        in_specs=[
            pl.BlockSpec((W, NBLK), lambda i: (0, i)),
            pl.BlockSpec((2, DEG, W, NBLK), lambda i: (0, 0, 0, i)),
            pl.BlockSpec((DEG, W, NBLK), lambda i: (0, 0, i)),
            pl.BlockSpec((NBLK, NCB * 24),
                         lambda i: (jnp.minimum(i, K // NBLK - 1), 0)),
        ],
        out_specs=[
            pl.BlockSpec((W, NBLK), lambda i: (0, i)),
            pl.BlockSpec((W, NCB * 24), lambda i: (0, 0)),
        ],
        out_shape=[
            jax.ShapeDtypeStruct((W, N), _f32),
            jax.ShapeDtypeStruct((W, NCB * 24), _f32),
        ],
    )(x, g, m, a)


def _deinterleave(llr_scr, perm_inv):
    @functools.partial(
        pl.kernel,
        out_type=jax.ShapeDtypeStruct((W * N,), _f32),
        mesh=_mesh,
        compiler_params=_sc_params,
        scratch_types=[
            pltpu.VMEM((NTOT,), _f32),
            pltpu.VMEM((HALF,), jnp.int32),
            pltpu.VMEM((HALF,), _f32),
        ],
    )
    def k(llr_ref, perm_ref, x_ref, tbl_v, idx_v, obuf_v):
        w = lax.axis_index("s")
        h = lax.axis_index("c")
        b = w // NCB
        cb = w % NCB
        pltpu.sync_copy(llr_ref.at[pl.ds(b * NTOT, NTOT)], tbl_v)
        pltpu.sync_copy(perm_ref.at[pl.ds(cb * N + h * HALF, HALF)], idx_v)

        unroll = 16
        def body(j, carry):
            for u in range(unroll):
                o = (j * unroll + u) * 16
                ii = idx_v[pl.ds(o, 16)]
                obuf_v[pl.ds(o, 16)] = plsc.load_gather(tbl_v, [ii])
            return carry

        lax.fori_loop(0, HALF // 16 // unroll, body, 0)
        pltpu.sync_copy(obuf_v, x_ref.at[pl.ds(w * N + h * HALF, HALF)])

    return k(llr_scr.reshape(-1), perm_inv).reshape(W, N)


def _cn_exchange(m, cn3):
    @functools.partial(
        pl.kernel,
        out_type=jax.ShapeDtypeStruct((2 * DEG * W * N,), _f32),
        mesh=_mesh,
        compiler_params=_sc_params,
        scratch_types=[
            pltpu.VMEM((E,), _f32),
            pltpu.VMEM((E,), jnp.int32),
            pltpu.VMEM((M,), _f32),
            pltpu.SemaphoreType.DMA,
        ],
    )
    def k(m_ref, cn_ref, g_ref, dat_v, idx_v, tbl_v, sem):
        q = lax.axis_index("c")
        w = lax.axis_index("s")
        cops = []
        for d in range(DEG):
            off = (d * W + w) * N
            cops.append(pltpu.async_copy(
                m_ref.at[pl.ds(off, N)], dat_v.at[pl.ds(d * N, N)], sem))
        cops.append(pltpu.async_copy(cn_ref, idx_v, sem))

        zunroll = 16
        def zbody(j, carry):
            for u in range(zunroll):
                tbl_v[pl.ds((j * zunroll + u) * 16, 16)] = jnp.zeros((16,), _f32)
            return carry

        lax.fori_loop(0, M // 16 // zunroll, zbody, 0)
        for c in cops:
            c.wait()

        is_mag = q == 0
        one = jnp.ones((16,), _f32)
        zero = jnp.zeros((16,), _f32)

        unroll = 8
        def sbody(j, carry):
            os = [(j * unroll + u) * 16 for u in range(unroll)]
            iis = [idx_v[pl.ds(o, 16)] for o in os]
            mms = [dat_v[pl.ds(o, 16)] for o in os]
            vvs = [jnp.where(is_mag, jnp.abs(mm),
                             jnp.where(mm < 0.0, one, zero)) for mm in mms]
            for ii, vv in zip(iis, vvs):
                plsc.addupdate_scatter(tbl_v, [ii], vv)
            return carry

        lax.fori_loop(0, E // 16 // unroll, sbody, 0)

        ocops = []
        for d in range(DEG):
            def gbody(j, carry, d=d):
                os = [d * N + (j * unroll + u) * 16 for u in range(unroll)]
                iis = [idx_v[pl.ds(o, 16)] for o in os]
                ggs = [plsc.load_gather(tbl_v, [ii]) for ii in iis]
                for o, gg in zip(os, ggs):
                    dat_v[pl.ds(o, 16)] = gg
                return carry

            lax.fori_loop(0, N // 16 // unroll, gbody, 0)
            off = ((q * DEG + d) * W + w) * N
            ocops.append(pltpu.async_copy(
                dat_v.at[pl.ds(d * N, N)], g_ref.at[pl.ds(off, N)], sem))
        for c in ocops:
            c.wait()

    return k(m.reshape(-1), cn3).reshape(2, DEG, W, N)


def kernel(inputs, scr_bits, crc_R, perm_inv, edge_vn, edge_cn):
    del edge_vn
    llr = inputs.astype(_f32).reshape(B, NTOT)
    scr = scr_bits.astype(_f32).reshape(1, NTOT)
    cn3 = edge_cn.reshape(N, DEG).T.reshape(E).astype(jnp.int32)

    llr_scr = _descramble(llr, scr)
    x = _deinterleave(llr_scr, perm_inv.astype(jnp.int32))
    m = _first_messages(x)
    for _ in range(NITER - 1):
        g = _cn_exchange(m, cn3)
        m = _bp_step(x, g, m)
    g = _cn_exchange(m, cn3)

    r4 = crc_R.astype(_f32).reshape(NCB, KI, 24)
    a = jnp.transpose(r4, (1, 0, 2)).reshape(KI, NCB * 24)
    a = jnp.pad(a, ((0, K - KI), (0, 0)))

    bits, gm = _finalize(x, g, m, a)

    gmr = gm.reshape(B, NCB, NCB, 24)
    syn = gmr[:, 0, 0, :] + gmr[:, 1, 1, :] + gmr[:, 2, 2, :] + gmr[:, 3, 3, :]
    syn = syn - 2.0 * jnp.floor(syn * 0.5)
    tb_crc_status = jnp.all(syn < 0.5, axis=-1)
    u_hat = bits[:, :KI].reshape(B, NCB * KI)[:, :TBS]
    return u_hat, tb_crc_status
